# baseline (speedup 1.0000x reference)
import functools

import jax
import jax.numpy as jnp
from jax import lax
from jax.experimental import pallas as pl
from jax.experimental.pallas import tpu as pltpu
from jax.experimental.pallas import tpu_sc as plsc

try:
    _info = plsc.get_sparse_core_info()
    _NC, _NS, _L = _info.num_cores, _info.num_subcores, _info.num_lanes
except Exception:
    _NC, _NS, _L = 2, 16, 16
_NW = _NC * _NS


@functools.partial(jax.jit, static_argnames=("n_chunks", "chunk", "halves", "d"))
def _embed(idx2d, token_table, pos3d, *, n_chunks, chunk, halves, d):
    chunks_per_w = n_chunks // _NW
    mesh = plsc.VectorSubcoreMesh(core_axis_name="c", subcore_axis_name="s")

    @functools.partial(
        pl.kernel,
        mesh=mesh,
        out_type=jax.ShapeDtypeStruct((n_chunks, chunk, d), jnp.float32),
        scratch_types=[
            pltpu.VMEM((chunks_per_w, chunk), jnp.int32),
            pltpu.VMEM((halves, chunk, d), jnp.float32),
            pltpu.VMEM((chunk, d), jnp.float32),
            pltpu.SemaphoreType.DMA,
        ],
        compiler_params=pltpu.CompilerParams(use_tc_tiling_on_sc=False),
    )
    def body(idx_hbm, tok_hbm, pos_hbm, out_hbm, idx_v, pos_v, rows_v, sem):
        wid = lax.axis_index("s") * _NC + lax.axis_index("c")
        c0 = wid * chunks_per_w
        pltpu.sync_copy(pos_hbm, pos_v)
        pltpu.sync_copy(idx_hbm.at[pl.ds(c0, chunks_per_w)], idx_v)

        def chunk_step(j, carry):
            pltpu.async_copy(tok_hbm.at[idx_v.at[j]], rows_v, sem).wait()
            half = lax.rem(c0 + j, halves)

            def add_row(r, carry2):
                for c in range(d // _L):
                    sl = pl.ds(c * _L, _L)
                    rows_v[r, sl] = rows_v[r, sl] + pos_v[half, r, sl]
                return carry2

            lax.fori_loop(0, chunk, add_row, 0, unroll=2)
            pltpu.sync_copy(rows_v, out_hbm.at[c0 + j])
            return carry

        lax.fori_loop(0, chunks_per_w, chunk_step, 0)

    return body(idx2d, token_table, pos3d)


def kernel(inputs, token_table, pos_table):
    b, s = inputs.shape
    v, d = token_table.shape
    halves = 1
    while s // halves > 128:
        halves *= 2
    chunk = s // halves
    n_chunks = b * halves
    assert s % halves == 0 and n_chunks % _NW == 0 and d % _L == 0

    idx2d = inputs.reshape(n_chunks, chunk).astype(jnp.int32)
    pos3d = pos_table.reshape(halves, chunk, d)
    out = _embed(idx2d, token_table, pos3d,
                 n_chunks=n_chunks, chunk=chunk, halves=halves, d=d)
    return out.reshape(b, s, d)


# speedup vs baseline: 1.0017x; 1.0017x over previous
import functools

import jax
import jax.numpy as jnp
from jax import lax
from jax.experimental import pallas as pl
from jax.experimental.pallas import tpu as pltpu
from jax.experimental.pallas import tpu_sc as plsc

try:
    _info = plsc.get_sparse_core_info()
    _NC, _NS, _L = _info.num_cores, _info.num_subcores, _info.num_lanes
except Exception:
    _NC, _NS, _L = 2, 16, 16
_NW = _NC * _NS


@functools.partial(jax.jit, static_argnames=("n_chunks", "chunk", "halves", "d"))
def _embed(idx2d, token_table, pos3d, *, n_chunks, chunk, halves, d):
    chunks_per_w = n_chunks // _NW
    mesh = plsc.VectorSubcoreMesh(core_axis_name="c", subcore_axis_name="s")

    @functools.partial(
        pl.kernel,
        mesh=mesh,
        out_type=jax.ShapeDtypeStruct((n_chunks * chunk, d), jnp.float32),
        scratch_types=[
            pltpu.VMEM((chunks_per_w, chunk), jnp.int32),
            pltpu.VMEM((halves, chunk, d), jnp.float32),
            pltpu.VMEM((chunk, d), jnp.float32),
            pltpu.SemaphoreType.DMA,
        ],
        compiler_params=pltpu.CompilerParams(use_tc_tiling_on_sc=False),
    )
    def body(idx_hbm, tok_hbm, pos_hbm, out_hbm, idx_v, pos_v, rows_v, sem):
        wid = lax.axis_index("s") * _NC + lax.axis_index("c")
        c0 = wid * chunks_per_w
        pltpu.sync_copy(pos_hbm, pos_v)
        pltpu.sync_copy(idx_hbm.at[pl.ds(c0, chunks_per_w)], idx_v)

        def chunk_step(j, carry):
            pltpu.async_copy(tok_hbm.at[idx_v.at[j]], rows_v, sem).wait()
            half = lax.rem(c0 + j, halves)

            def add_row(r, carry2):
                for c in range(d // _L):
                    sl = pl.ds(c * _L, _L)
                    rows_v[r, sl] = rows_v[r, sl] + pos_v[half, r, sl]
                return carry2

            lax.fori_loop(0, chunk, add_row, 0, unroll=2)
            pltpu.sync_copy(rows_v, out_hbm.at[pl.ds((c0 + j) * chunk, chunk)])
            return carry

        lax.fori_loop(0, chunks_per_w, chunk_step, 0)

    return body(idx2d, token_table, pos3d)


def kernel(inputs, token_table, pos_table):
    b, s = inputs.shape
    v, d = token_table.shape
    halves = 1
    while s // halves > 128:
        halves *= 2
    chunk = s // halves
    n_chunks = b * halves
    assert s % halves == 0 and n_chunks % _NW == 0 and d % _L == 0

    idx2d = inputs.reshape(n_chunks, chunk).astype(jnp.int32)
    pos3d = pos_table.reshape(halves, chunk, d)
    out = _embed(idx2d, token_table, pos3d,
                 n_chunks=n_chunks, chunk=chunk, halves=halves, d=d)
    return out.reshape(b, s, d)


# speedup vs baseline: 1.0975x; 1.0957x over previous
import functools

import jax
import jax.numpy as jnp
from jax import lax
from jax.experimental import pallas as pl
from jax.experimental.pallas import tpu as pltpu
from jax.experimental.pallas import tpu_sc as plsc

try:
    _info = plsc.get_sparse_core_info()
    _NC, _NS, _L = _info.num_cores, _info.num_subcores, _info.num_lanes
except Exception:
    _NC, _NS, _L = 2, 16, 16
_NW = _NC * _NS

_CHUNK = 128


def _dyn_gather(vec, idx):
    return lax.gather(
        vec, idx[:, None],
        dimension_numbers=lax.GatherDimensionNumbers(
            offset_dims=(), collapsed_slice_dims=(0,), start_index_map=(0,)),
        slice_sizes=(1,),
        mode=lax.GatherScatterMode.PROMISE_IN_BOUNDS)


@functools.partial(jax.jit, static_argnames=("seq", "d"))
def _embed(idx1d, tbl2, pos2d, *, seq, d):
    total = idx1d.shape[0]
    n_chunks = total // _CHUNK
    chunks_per_w = n_chunks // _NW
    per_w = chunks_per_w * _CHUNK
    mesh = plsc.VectorSubcoreMesh(core_axis_name="c", subcore_axis_name="s")

    @functools.partial(
        pl.kernel,
        mesh=mesh,
        out_type=jax.ShapeDtypeStruct((total, d), jnp.float32),
        scratch_types=[
            pltpu.VMEM((per_w,), jnp.int32),
            pltpu.VMEM((per_w,), jnp.int32),
            pltpu.VMEM((seq, d), jnp.float32),
            pltpu.VMEM((2, _CHUNK, 2 * d), jnp.float32),
            pltpu.VMEM((2, _CHUNK, d), jnp.float32),
            pltpu.SemaphoreType.DMA,
            pltpu.SemaphoreType.DMA,
        ],
        compiler_params=pltpu.CompilerParams(use_tc_tiling_on_sc=True,
                                             needs_layout_passes=False),
    )
    def body(idx_hbm, tbl_hbm, pos_hbm, out_hbm,
             row_v, off_v, pos_v, g_v, rows_v, gsem, osem):
        wid = lax.axis_index("s") * _NC + lax.axis_index("c")
        c0 = wid * chunks_per_w
        pltpu.sync_copy(pos_hbm, pos_v)
        pltpu.sync_copy(idx_hbm.at[pl.ds(wid * per_w, per_w)], row_v)

        def split_ids(i, carry):
            sl = pl.ds(i * _L, _L)
            v = row_v[sl]
            off_v[sl] = (v & 1) * d
            row_v[sl] = v >> 1
            return carry

        lax.fori_loop(0, per_w // _L, split_ids, 0, unroll=8)

        def fire_gather(j, buf):
            pltpu.async_copy(
                tbl_hbm.at[row_v.at[pl.ds(j * _CHUNK, _CHUNK)]],
                g_v.at[buf], gsem)

        def wait_gather(j, buf):
            pltpu.make_async_copy(
                tbl_hbm.at[row_v.at[pl.ds(j * _CHUNK, _CHUNK)]],
                g_v.at[buf], gsem).wait()

        def out_slice(j):
            return out_hbm.at[pl.ds((c0 + j) * _CHUNK, _CHUNK)]

        fire_gather(0, 0)
        fire_gather(1, 1)

        def step(j, carry):
            buf = lax.rem(j, 2)
            wait_gather(j, buf)

            @pl.when(j >= 2)
            def _():
                pltpu.make_async_copy(rows_v.at[buf], out_slice(j - 2),
                                      osem).wait()

            base = (c0 + j) * _CHUNK

            def do_row(r, carry2):
                hoff = _dyn_gather(
                    off_v[pl.ds(j * _CHUNK + (r // _L) * _L, _L)],
                    jnp.full((_L,), lax.rem(r, _L), jnp.int32))
                s = lax.rem(base + r, seq)
                rsplat = jnp.full((_L,), r, jnp.int32)
                for c in range(d // _L):
                    sl = pl.ds(c * _L, _L)
                    col = hoff + (c * _L + lax.iota(jnp.int32, _L))
                    val = plsc.load_gather(g_v.at[buf], [rsplat, col])
                    rows_v[buf, r, sl] = val + pos_v[s, sl]
                return carry2

            lax.fori_loop(0, _CHUNK, do_row, 0, unroll=2)
            pltpu.async_copy(rows_v.at[buf], out_slice(j), osem)

            @pl.when(j + 2 < chunks_per_w)
            def _():
                fire_gather(j + 2, buf)
            return carry

        lax.fori_loop(0, chunks_per_w, step, 0)
        pltpu.make_async_copy(rows_v.at[0], out_slice(chunks_per_w - 2),
                              osem).wait()
        pltpu.make_async_copy(rows_v.at[1], out_slice(chunks_per_w - 1),
                              osem).wait()

    return body(idx1d, tbl2, pos2d)


def kernel(inputs, token_table, pos_table):
    b, s = inputs.shape
    v, d = token_table.shape
    total = b * s
    n_chunks = total // _CHUNK
    assert total % _CHUNK == 0 and n_chunks % _NW == 0
    assert (n_chunks // _NW) % 2 == 0 and v % 2 == 0 and d % _L == 0

    idx1d = inputs.reshape(total).astype(jnp.int32)
    tbl2 = token_table.reshape(v // 2, 2 * d)
    out = _embed(idx1d, tbl2, pos_table, seq=s, d=d)
    return out.reshape(b, s, d)
